# baseline (speedup 1.0000x reference)
import functools

import jax
import jax.numpy as jnp
from jax import lax
from jax.experimental import pallas as pl
from jax.experimental.pallas import tpu as pltpu

NB = 256
LANE = 128
K5 = 5
H1, W1 = 28, 28
HP1, WP1 = 14, 14
WP1P = 16
H2, W2 = 10, 10
HP2, WP2 = 5, 5
C1, C2 = 6, 16
M1 = 2 * C1 * WP1P
K1 = K5 * 32
M2 = 2 * C2 * WP2
K2 = C1 * K5 * WP1P
KF = HP2 * C2 * WP2
_VMEM_LIMIT = 64 * 1024 * 1024


def _lenet_kernel(xT_ref, a1_ref, b1_ref, a2_ref, b2_ref,
                  w3_ref, b3_ref, w4_ref, b4_ref, w5_ref, b5_ref,
                  o_ref,
                  p1_scr, p2_scr):
    f32 = jnp.float32
    bf16 = jnp.bfloat16

    for i in range(HP1):
        xa = xT_ref[2 * i * 32:2 * i * 32 + K1, :]
        xb = xT_ref[(2 * i + 1) * 32:(2 * i + 1) * 32 + K1, :]
        ya = jnp.dot(a1_ref[...], xa, preferred_element_type=f32)
        yb = jnp.dot(a1_ref[...], xb, preferred_element_type=f32)
        m = jnp.maximum(ya, yb)
        pooled = jnp.maximum(m[0:M1 // 2], m[M1 // 2:M1])
        row = jnp.maximum(pooled + b1_ref[...], 0.0)
        p1_scr[:, i, :, :] = row.astype(bf16).reshape(C1, WP1P, NB)

    for i in range(HP2):
        x2a = p1_scr[:, 2 * i:2 * i + K5, :, :].reshape(K2, NB)
        x2b = p1_scr[:, 2 * i + 1:2 * i + 1 + K5, :, :].reshape(K2, NB)
        ya = jnp.dot(a2_ref[...], x2a, preferred_element_type=f32)
        yb = jnp.dot(a2_ref[...], x2b, preferred_element_type=f32)
        m = jnp.maximum(ya, yb)
        pooled = jnp.maximum(m[0:M2 // 2], m[M2 // 2:M2])
        prow = jnp.maximum(pooled + b2_ref[...], 0.0)
        p2_scr[i] = prow.astype(bf16)

    xf = p2_scr[...].reshape(KF, NB)
    accf = jnp.dot(w3_ref[...], xf, preferred_element_type=f32)
    h3 = jnp.maximum(accf + b3_ref[...], 0.0).astype(bf16)
    h4 = jnp.maximum(jnp.dot(w4_ref[...], h3, preferred_element_type=f32)
                     + b4_ref[...], 0.0).astype(bf16)
    ot = lax.dot_general(h4, w5_ref[...], (((0,), (1,)), ((), ())),
                         preferred_element_type=f32)
    o_ref[...] = ot + b5_ref[...]


@jax.jit
def _lenet_forward(x, c1_w, c1_b, c2_w, c2_b, fc1_w, fc1_b,
                   fc2_w, fc2_b, fc3_w, fc3_b):
    n = x.shape[0]
    n_pad = ((n + NB - 1) // NB) * NB
    xp = jnp.pad(x, ((0, n_pad - n), (0, 0), (2, 2), (2, 2)))
    xT = jnp.transpose(xp.reshape(n_pad, 32 * 32).astype(jnp.bfloat16), (1, 0))

    f32 = jnp.float32
    bf16 = jnp.bfloat16
    w1p = jnp.transpose(c1_w[:, :C1].reshape(K5, K5, C1), (2, 0, 1))
    wv1 = 2 * jnp.arange(WP1P)[None, :] + jnp.arange(2)[:, None]
    band1 = (jnp.arange(32)[None, None, :, None]
             - wv1[:, :, None, None]
             == jnp.arange(K5)[None, None, None, :]).astype(f32)
    a1 = jnp.einsum('ohq,pwvq->powhv', w1p, band1).reshape(M1, K1).astype(bf16)
    b1b = jnp.broadcast_to(
        jnp.repeat(c1_b[0, :C1], WP1P)[:, None], (M1 // 2, NB)).astype(f32)

    w2p = jnp.transpose(c2_w[:, :C1, :C2].astype(f32).reshape(K5, K5, C1, C2),
                        (3, 2, 0, 1))
    wv2 = 2 * jnp.arange(WP2)[None, :] + jnp.arange(2)[:, None]
    band2 = (jnp.arange(WP1P)[None, None, :, None]
             - wv2[:, :, None, None]
             == jnp.arange(K5)[None, None, None, :]).astype(f32)
    a2 = jnp.einsum('ochq,pwvq->powchv', w2p, band2).reshape(M2, K2).astype(bf16)
    b2b = jnp.broadcast_to(
        jnp.repeat(c2_b[0, :C2], WP2)[:, None], (M2 // 2, NB)).astype(f32)

    w3big = jnp.transpose(fc1_w.reshape(HP2, WP2, LANE, LANE)[:, :, :C2, :],
                          (3, 0, 2, 1)).reshape(LANE, KF).astype(bf16)
    w4 = jnp.swapaxes(fc2_w, 0, 1).astype(bf16)
    w5 = jnp.swapaxes(fc3_w, 0, 1).astype(bf16)
    b3 = jnp.broadcast_to(jnp.swapaxes(fc1_b, 0, 1), (LANE, NB)).astype(f32)
    b4 = jnp.broadcast_to(jnp.swapaxes(fc2_b, 0, 1), (LANE, NB)).astype(f32)
    b5 = fc3_b.astype(f32)

    grid = (n_pad // NB,)
    out = pl.pallas_call(
        _lenet_kernel,
        out_shape=jax.ShapeDtypeStruct((n_pad, LANE), f32),
        grid=grid,
        in_specs=[
            pl.BlockSpec((32 * 32, NB), lambda i: (0, i)),
            pl.BlockSpec((M1, K1), lambda i: (0, 0)),
            pl.BlockSpec((M1 // 2, NB), lambda i: (0, 0)),
            pl.BlockSpec((M2, K2), lambda i: (0, 0)),
            pl.BlockSpec((M2 // 2, NB), lambda i: (0, 0)),
            pl.BlockSpec((LANE, KF), lambda i: (0, 0)),
            pl.BlockSpec((LANE, NB), lambda i: (0, 0)),
            pl.BlockSpec((LANE, LANE), lambda i: (0, 0)),
            pl.BlockSpec((LANE, NB), lambda i: (0, 0)),
            pl.BlockSpec((LANE, LANE), lambda i: (0, 0)),
            pl.BlockSpec((1, LANE), lambda i: (0, 0)),
        ],
        out_specs=pl.BlockSpec((NB, LANE), lambda i: (i, 0)),
        scratch_shapes=[
            pltpu.VMEM((C1, HP1, WP1P, NB), jnp.bfloat16),
            pltpu.VMEM((HP2, M2 // 2, NB), jnp.bfloat16),
        ],
        compiler_params=pltpu.CompilerParams(
            dimension_semantics=("parallel",),
            vmem_limit_bytes=_VMEM_LIMIT),
    )(xT, a1, b1b, a2, b2b, w3big, b3, w4, b4, w5, b5)
    return out[:n, :10]


def kernel(x, c1_w, c1_b, c2_w, c2_b, fc1_w, fc1_b, fc2_w, fc2_b, fc3_w, fc3_b):
    return _lenet_forward(x, c1_w, c1_b, c2_w, c2_b, fc1_w, fc1_b,
                          fc2_w, fc2_b, fc3_w, fc3_b)


# speedup vs baseline: 1.0578x; 1.0578x over previous
import functools

import jax
import jax.numpy as jnp
from jax import lax
from jax.experimental import pallas as pl
from jax.experimental.pallas import tpu as pltpu

NB = 512
LANE = 128
K5 = 5
H1, W1 = 28, 28
HP1, WP1 = 14, 14
WP1P = 16
H2, W2 = 10, 10
HP2, WP2 = 5, 5
C1, C2 = 6, 16
M1 = 2 * C1 * WP1P
K1 = K5 * 32
M2 = 2 * C2 * WP2
K2 = C1 * K5 * WP1P
KF = HP2 * C2 * WP2
_VMEM_LIMIT = 64 * 1024 * 1024


def _lenet_kernel(xT_ref, a1_ref, b1_ref, a2_ref, b2_ref,
                  w3_ref, b3_ref, w4_ref, b4_ref, w5_ref, b5_ref,
                  o_ref,
                  p1_scr, p2_scr):
    f32 = jnp.float32
    bf16 = jnp.bfloat16

    for i in range(HP1):
        xa = xT_ref[2 * i * 32:2 * i * 32 + K1, :]
        xb = xT_ref[(2 * i + 1) * 32:(2 * i + 1) * 32 + K1, :]
        ya = jnp.dot(a1_ref[...], xa, preferred_element_type=f32)
        yb = jnp.dot(a1_ref[...], xb, preferred_element_type=f32)
        m = jnp.maximum(ya, yb)
        pooled = jnp.maximum(m[0:M1 // 2], m[M1 // 2:M1])
        row = jnp.maximum(pooled + b1_ref[...], 0.0)
        p1_scr[:, i, :, :] = row.astype(bf16).reshape(C1, WP1P, NB)

    for i in range(HP2):
        x2a = p1_scr[:, 2 * i:2 * i + K5, :, :].reshape(K2, NB)
        x2b = p1_scr[:, 2 * i + 1:2 * i + 1 + K5, :, :].reshape(K2, NB)
        ya = jnp.dot(a2_ref[...], x2a, preferred_element_type=f32)
        yb = jnp.dot(a2_ref[...], x2b, preferred_element_type=f32)
        m = jnp.maximum(ya, yb)
        pooled = jnp.maximum(m[0:M2 // 2], m[M2 // 2:M2])
        prow = jnp.maximum(pooled + b2_ref[...], 0.0)
        p2_scr[i] = prow.astype(bf16)

    xf = p2_scr[...].reshape(KF, NB)
    accf = jnp.dot(w3_ref[...], xf, preferred_element_type=f32)
    h3 = jnp.maximum(accf + b3_ref[...], 0.0).astype(bf16)
    h4 = jnp.maximum(jnp.dot(w4_ref[...], h3, preferred_element_type=f32)
                     + b4_ref[...], 0.0).astype(bf16)
    o_ref[...] = (jnp.dot(w5_ref[...], h4, preferred_element_type=f32)
                  + b5_ref[...])


@jax.jit
def _lenet_forward(x, c1_w, c1_b, c2_w, c2_b, fc1_w, fc1_b,
                   fc2_w, fc2_b, fc3_w, fc3_b):
    n = x.shape[0]
    n_pad = ((n + NB - 1) // NB) * NB
    xp = jnp.pad(x, ((0, n_pad - n), (0, 0), (2, 2), (2, 2)))
    xT = jnp.transpose(xp.reshape(n_pad, 32 * 32).astype(jnp.bfloat16), (1, 0))

    f32 = jnp.float32
    bf16 = jnp.bfloat16
    w1p = jnp.transpose(c1_w[:, :C1].reshape(K5, K5, C1), (2, 0, 1))
    wv1 = 2 * jnp.arange(WP1P)[None, :] + jnp.arange(2)[:, None]
    band1 = (jnp.arange(32)[None, None, :, None]
             - wv1[:, :, None, None]
             == jnp.arange(K5)[None, None, None, :]).astype(f32)
    a1 = jnp.einsum('ohq,pwvq->powhv', w1p, band1).reshape(M1, K1).astype(bf16)
    b1b = jnp.broadcast_to(
        jnp.repeat(c1_b[0, :C1], WP1P)[:, None], (M1 // 2, NB)).astype(f32)

    w2p = jnp.transpose(c2_w[:, :C1, :C2].astype(f32).reshape(K5, K5, C1, C2),
                        (3, 2, 0, 1))
    wv2 = 2 * jnp.arange(WP2)[None, :] + jnp.arange(2)[:, None]
    band2 = (jnp.arange(WP1P)[None, None, :, None]
             - wv2[:, :, None, None]
             == jnp.arange(K5)[None, None, None, :]).astype(f32)
    a2 = jnp.einsum('ochq,pwvq->powchv', w2p, band2).reshape(M2, K2).astype(bf16)
    b2b = jnp.broadcast_to(
        jnp.repeat(c2_b[0, :C2], WP2)[:, None], (M2 // 2, NB)).astype(f32)

    w3big = jnp.transpose(fc1_w.reshape(HP2, WP2, LANE, LANE)[:, :, :C2, :],
                          (3, 0, 2, 1)).reshape(LANE, KF).astype(bf16)
    w4 = jnp.swapaxes(fc2_w, 0, 1).astype(bf16)
    w5 = jnp.swapaxes(fc3_w, 0, 1).astype(bf16)
    b3 = jnp.broadcast_to(jnp.swapaxes(fc1_b, 0, 1), (LANE, NB)).astype(f32)
    b4 = jnp.broadcast_to(jnp.swapaxes(fc2_b, 0, 1), (LANE, NB)).astype(f32)
    b5 = jnp.broadcast_to(jnp.swapaxes(fc3_b, 0, 1), (LANE, NB)).astype(f32)

    grid = (n_pad // NB,)
    out = pl.pallas_call(
        _lenet_kernel,
        out_shape=jax.ShapeDtypeStruct((LANE, n_pad), f32),
        grid=grid,
        in_specs=[
            pl.BlockSpec((32 * 32, NB), lambda i: (0, i)),
            pl.BlockSpec((M1, K1), lambda i: (0, 0)),
            pl.BlockSpec((M1 // 2, NB), lambda i: (0, 0)),
            pl.BlockSpec((M2, K2), lambda i: (0, 0)),
            pl.BlockSpec((M2 // 2, NB), lambda i: (0, 0)),
            pl.BlockSpec((LANE, KF), lambda i: (0, 0)),
            pl.BlockSpec((LANE, NB), lambda i: (0, 0)),
            pl.BlockSpec((LANE, LANE), lambda i: (0, 0)),
            pl.BlockSpec((LANE, NB), lambda i: (0, 0)),
            pl.BlockSpec((LANE, LANE), lambda i: (0, 0)),
            pl.BlockSpec((LANE, NB), lambda i: (0, 0)),
        ],
        out_specs=pl.BlockSpec((LANE, NB), lambda i: (0, i)),
        scratch_shapes=[
            pltpu.VMEM((C1, HP1, WP1P, NB), jnp.bfloat16),
            pltpu.VMEM((HP2, M2 // 2, NB), jnp.bfloat16),
        ],
        compiler_params=pltpu.CompilerParams(
            dimension_semantics=("parallel",),
            vmem_limit_bytes=_VMEM_LIMIT),
    )(xT, a1, b1b, a2, b2b, w3big, b3, w4, b4, w5, b5)
    return jnp.transpose(out[:10, :n], (1, 0))


def kernel(x, c1_w, c1_b, c2_w, c2_b, fc1_w, fc1_b, fc2_w, fc2_b, fc3_w, fc3_b):
    return _lenet_forward(x, c1_w, c1_b, c2_w, c2_b, fc1_w, fc1_b,
                          fc2_w, fc2_b, fc3_w, fc3_b)


# speedup vs baseline: 1.0981x; 1.0381x over previous
import functools

import jax
import jax.numpy as jnp
from jax import lax
from jax.experimental import pallas as pl
from jax.experimental.pallas import tpu as pltpu

NB = 1024
LANE = 128
K5 = 5
H1, W1 = 28, 28
HP1, WP1 = 14, 14
WP1P = 16
H2, W2 = 10, 10
HP2, WP2 = 5, 5
C1, C2 = 6, 16
M1 = 2 * C1 * WP1P
K1 = K5 * 32
M2 = 2 * C2 * WP2
K2 = C1 * K5 * WP1P
KF = HP2 * C2 * WP2
_VMEM_LIMIT = 64 * 1024 * 1024


def _lenet_kernel(xT_ref, a1_ref, b1_ref, a2_ref, b2_ref,
                  w3_ref, b3_ref, w4_ref, b4_ref, w5_ref, b5_ref,
                  o_ref,
                  p1_scr, p2_scr):
    f32 = jnp.float32
    bf16 = jnp.bfloat16

    for i in range(HP1):
        xa = xT_ref[2 * i * 32:2 * i * 32 + K1, :]
        xb = xT_ref[(2 * i + 1) * 32:(2 * i + 1) * 32 + K1, :]
        ya = jnp.dot(a1_ref[...], xa, preferred_element_type=f32)
        yb = jnp.dot(a1_ref[...], xb, preferred_element_type=f32)
        m = jnp.maximum(ya, yb)
        pooled = jnp.maximum(m[0:M1 // 2], m[M1 // 2:M1])
        row = jnp.maximum(pooled + b1_ref[...], 0.0)
        p1_scr[:, i, :, :] = row.astype(bf16).reshape(C1, WP1P, NB)

    for i in range(HP2):
        x2a = p1_scr[:, 2 * i:2 * i + K5, :, :].reshape(K2, NB)
        x2b = p1_scr[:, 2 * i + 1:2 * i + 1 + K5, :, :].reshape(K2, NB)
        ya = jnp.dot(a2_ref[...], x2a, preferred_element_type=f32)
        yb = jnp.dot(a2_ref[...], x2b, preferred_element_type=f32)
        m = jnp.maximum(ya, yb)
        pooled = jnp.maximum(m[0:M2 // 2], m[M2 // 2:M2])
        prow = jnp.maximum(pooled + b2_ref[...], 0.0)
        p2_scr[i] = prow.astype(bf16)

    xf = p2_scr[...].reshape(KF, NB)
    accf = jnp.dot(w3_ref[...], xf, preferred_element_type=f32)
    h3 = jnp.maximum(accf + b3_ref[...], 0.0).astype(bf16)
    h4 = jnp.maximum(jnp.dot(w4_ref[...], h3, preferred_element_type=f32)
                     + b4_ref[...], 0.0).astype(bf16)
    o_ref[...] = (jnp.dot(w5_ref[...], h4, preferred_element_type=f32)
                  + b5_ref[...])


@jax.jit
def _lenet_forward(x, c1_w, c1_b, c2_w, c2_b, fc1_w, fc1_b,
                   fc2_w, fc2_b, fc3_w, fc3_b):
    n = x.shape[0]
    n_pad = ((n + NB - 1) // NB) * NB
    xp = jnp.pad(x, ((0, n_pad - n), (0, 0), (2, 2), (2, 2)))
    xT = jnp.transpose(xp.reshape(n_pad, 32 * 32).astype(jnp.bfloat16), (1, 0))

    f32 = jnp.float32
    bf16 = jnp.bfloat16
    w1p = jnp.transpose(c1_w[:, :C1].reshape(K5, K5, C1), (2, 0, 1))
    wv1 = 2 * jnp.arange(WP1P)[None, :] + jnp.arange(2)[:, None]
    band1 = (jnp.arange(32)[None, None, :, None]
             - wv1[:, :, None, None]
             == jnp.arange(K5)[None, None, None, :]).astype(f32)
    a1 = jnp.einsum('ohq,pwvq->powhv', w1p, band1).reshape(M1, K1).astype(bf16)
    b1b = jnp.broadcast_to(
        jnp.repeat(c1_b[0, :C1], WP1P)[:, None], (M1 // 2, NB)).astype(f32)

    w2p = jnp.transpose(c2_w[:, :C1, :C2].astype(f32).reshape(K5, K5, C1, C2),
                        (3, 2, 0, 1))
    wv2 = 2 * jnp.arange(WP2)[None, :] + jnp.arange(2)[:, None]
    band2 = (jnp.arange(WP1P)[None, None, :, None]
             - wv2[:, :, None, None]
             == jnp.arange(K5)[None, None, None, :]).astype(f32)
    a2 = jnp.einsum('ochq,pwvq->powchv', w2p, band2).reshape(M2, K2).astype(bf16)
    b2b = jnp.broadcast_to(
        jnp.repeat(c2_b[0, :C2], WP2)[:, None], (M2 // 2, NB)).astype(f32)

    w3big = jnp.transpose(fc1_w.reshape(HP2, WP2, LANE, LANE)[:, :, :C2, :],
                          (3, 0, 2, 1)).reshape(LANE, KF).astype(bf16)
    w4 = jnp.swapaxes(fc2_w, 0, 1).astype(bf16)
    w5 = jnp.swapaxes(fc3_w, 0, 1).astype(bf16)
    b3 = jnp.broadcast_to(jnp.swapaxes(fc1_b, 0, 1), (LANE, NB)).astype(f32)
    b4 = jnp.broadcast_to(jnp.swapaxes(fc2_b, 0, 1), (LANE, NB)).astype(f32)
    b5 = jnp.broadcast_to(jnp.swapaxes(fc3_b, 0, 1), (LANE, NB)).astype(f32)

    grid = (n_pad // NB,)
    out = pl.pallas_call(
        _lenet_kernel,
        out_shape=jax.ShapeDtypeStruct((LANE, n_pad), f32),
        grid=grid,
        in_specs=[
            pl.BlockSpec((32 * 32, NB), lambda i: (0, i)),
            pl.BlockSpec((M1, K1), lambda i: (0, 0)),
            pl.BlockSpec((M1 // 2, NB), lambda i: (0, 0)),
            pl.BlockSpec((M2, K2), lambda i: (0, 0)),
            pl.BlockSpec((M2 // 2, NB), lambda i: (0, 0)),
            pl.BlockSpec((LANE, KF), lambda i: (0, 0)),
            pl.BlockSpec((LANE, NB), lambda i: (0, 0)),
            pl.BlockSpec((LANE, LANE), lambda i: (0, 0)),
            pl.BlockSpec((LANE, NB), lambda i: (0, 0)),
            pl.BlockSpec((LANE, LANE), lambda i: (0, 0)),
            pl.BlockSpec((LANE, NB), lambda i: (0, 0)),
        ],
        out_specs=pl.BlockSpec((LANE, NB), lambda i: (0, i)),
        scratch_shapes=[
            pltpu.VMEM((C1, HP1, WP1P, NB), jnp.bfloat16),
            pltpu.VMEM((HP2, M2 // 2, NB), jnp.bfloat16),
        ],
        compiler_params=pltpu.CompilerParams(
            dimension_semantics=("parallel",),
            vmem_limit_bytes=_VMEM_LIMIT),
    )(xT, a1, b1b, a2, b2b, w3big, b3, w4, b4, w5, b5)
    return jnp.transpose(out[:10, :n], (1, 0))


def kernel(x, c1_w, c1_b, c2_w, c2_b, fc1_w, fc1_b, fc2_w, fc2_b, fc3_w, fc3_b):
    return _lenet_forward(x, c1_w, c1_b, c2_w, c2_b, fc1_w, fc1_b,
                          fc2_w, fc2_b, fc3_w, fc3_b)
